# Initial kernel scaffold; baseline (speedup 1.0000x reference)
import functools

import jax
import jax.numpy as jnp
from jax import lax
from jax.experimental import pallas as pl
from jax.experimental.pallas import tpu as pltpu
from jax.experimental.pallas import tpu_sc as plsc

N_USERS = 25000
N_ITEMS = 25000
N_NODES = N_USERS + N_ITEMS
D = 64
DH = D // 2
NNZ = 800000
N_LAYERS = 3

NS = 16
NC = 2
STRIPE = N_NODES // NS
E_CHUNK = 1024
E_GRP = 128
GRPS = E_CHUNK // E_GRP
CHUNKS_PER_TILE = -(-NNZ // (NS * E_CHUNK))
EDGES_PER_TILE = CHUNKS_PER_TILE * E_CHUNK
NNZ_PAD = EDGES_PER_TILE * NS


_PRE_R = 2000


def _pre_body(t_ref, emb_ref, wa_ref, ba_ref, wp_ref, out_ref):
    ph = t_ref[:] * wa_ref[:] + ba_ref[:]
    lane = lax.broadcasted_iota(jnp.int32, ph.shape, 1)
    val = jnp.where(lane == 0, ph, jnp.sin(ph))
    res = lax.dot_general(val, wp_ref[:], (((1,), (1,)), ((), ())),
                          preferred_element_type=jnp.float32)
    out_ref[:] = res + emb_ref[:]


def _preamble(t_all, emb_all, wa, ba, wproj):
    nb = N_NODES // _PRE_R
    return pl.pallas_call(
        _pre_body,
        grid=(nb,),
        in_specs=[
            pl.BlockSpec((_PRE_R, 1), lambda i: (i, 0)),
            pl.BlockSpec((_PRE_R, D), lambda i: (i, 0)),
            pl.BlockSpec((1, D), lambda i: (0, 0)),
            pl.BlockSpec((1, D), lambda i: (0, 0)),
            pl.BlockSpec((D, D), lambda i: (0, 0)),
        ],
        out_specs=pl.BlockSpec((_PRE_R, D), lambda i: (i, 0)),
        out_shape=jax.ShapeDtypeStruct((N_NODES, D), jnp.float32),
    )(t_all, emb_all, wa, ba, wproj)


def _spmm_body(x0, cols2d, rows2d, vals2d, zstripe,
               o1, o2, o3,
               colv, colv2, rowv, valv, gath, acc, sem):
    c = lax.axis_index("c")
    s = lax.axis_index("s")
    col_off = c * N_NODES
    row0 = s * STRIPE
    ebase_row = s * (CHUNKS_PER_TILE * GRPS)

    def zero_stripe():
        pltpu.sync_copy(zstripe, acc.at[pl.ds(row0, STRIPE)])

    def run_layer(src, dst):
        def chunk(k, carry):
            rb = ebase_row + k * GRPS
            pltpu.sync_copy(cols2d.at[pl.ds(rb, GRPS)], colv)
            pltpu.sync_copy(rows2d.at[pl.ds(rb, GRPS)], rowv)
            pltpu.sync_copy(vals2d.at[pl.ds(rb, GRPS)], valv)
            for j in range(GRPS):
                for q in range(E_GRP // 16):
                    colv2[j, pl.ds(q * 16, 16)] = (
                        colv[j, pl.ds(q * 16, 16)] + col_off)
            descs = [pltpu.async_copy(src.at[colv2.at[j]], gath.at[j], sem)
                     for j in range(GRPS)]
            for d in descs:
                d.wait()
            for j in range(GRPS):
                def sbody(e, _, j=j):
                    sv = valv[j, e]
                    gath[j, e, pl.ds(0, 16)] = gath[j, e, pl.ds(0, 16)] * sv
                    gath[j, e, pl.ds(16, 16)] = gath[j, e, pl.ds(16, 16)] * sv
                    return 0
                lax.fori_loop(0, E_GRP, sbody, 0)
            for j in range(GRPS):
                pltpu.sync_copy(gath.at[j], acc.at[rowv.at[j]], add=True)
            return carry

        lax.fori_loop(0, CHUNKS_PER_TILE, chunk, 0)
        plsc.subcore_barrier()
        pltpu.sync_copy(acc.at[pl.ds(row0, STRIPE)],
                        dst.at[pl.ds(col_off + row0, STRIPE)])

    zero_stripe()
    plsc.subcore_barrier()
    run_layer(x0, o1)
    zero_stripe()
    plsc.subcore_barrier()
    run_layer(o1, o2)
    zero_stripe()
    plsc.subcore_barrier()
    run_layer(o2, o3)


def _spmm3(x0, cols2d, rows2d, vals2d, zstripe):
    mesh = plsc.VectorSubcoreMesh(core_axis_name="c", subcore_axis_name="s")
    xshape = jax.ShapeDtypeStruct((NC * N_NODES, DH), jnp.float32)
    f = pl.kernel(
        _spmm_body,
        out_type=(xshape, xshape, xshape),
        mesh=mesh,
        scratch_types=[
            pltpu.VMEM((GRPS, E_GRP), jnp.int32),
            pltpu.VMEM((GRPS, E_GRP), jnp.int32),
            pltpu.VMEM((GRPS, E_GRP), jnp.int32),
            pltpu.VMEM((GRPS, E_GRP), jnp.float32),
            pltpu.VMEM((GRPS, E_GRP, DH), jnp.float32),
            pltpu.VMEM_SHARED((N_NODES, DH), jnp.float32),
            pltpu.SemaphoreType.DMA,
        ],
    )
    return f(x0, cols2d, rows2d, vals2d, zstripe)


_M_R = 1000


def _mean_body(a, b, c, d, o):
    o[:] = (a[:] + b[:] + c[:] + d[:]) * 0.25


def _mean4(a, b, c, d):
    nb = a.shape[0] // _M_R
    spec = pl.BlockSpec((_M_R, 128), lambda i: (i, 0))
    return pl.pallas_call(
        _mean_body,
        grid=(nb,),
        in_specs=[spec] * 4,
        out_specs=spec,
        out_shape=jax.ShapeDtypeStruct(a.shape, jnp.float32),
    )(a, b, c, d)


def kernel(user_recency, item_recency, adj_vals, user_emb, item_emb,
           w0, b0, w, b, Wproj, edge_index):
    t_all = jnp.concatenate([user_recency, item_recency]).reshape(N_NODES, 1)
    emb_all = jnp.concatenate([user_emb, item_emb], axis=0)
    wa = jnp.concatenate([w0, w]).reshape(1, D)
    ba = jnp.concatenate([b0, b]).reshape(1, D)

    all_emb = _preamble(t_all, emb_all, wa, ba, Wproj)

    x0 = jnp.transpose(all_emb.reshape(N_NODES, NC, DH), (1, 0, 2))
    x0 = x0.reshape(NC * N_NODES, DH)

    rows = edge_index[0].astype(jnp.int32)
    cols = edge_index[1].astype(jnp.int32)
    pad = NNZ_PAD - NNZ
    ipad = jnp.zeros((pad,), jnp.int32)
    rows2d = jnp.concatenate([rows, ipad]).reshape(-1, E_GRP)
    cols2d = jnp.concatenate([cols, ipad]).reshape(-1, E_GRP)
    vals2d = jnp.concatenate([adj_vals, jnp.zeros((pad,), jnp.float32)])
    vals2d = vals2d.reshape(-1, E_GRP)
    zstripe = jnp.zeros((STRIPE, DH), jnp.float32)

    x1, x2, x3 = _spmm3(x0, cols2d, rows2d, vals2d, zstripe)

    flat = lambda v: v.reshape(N_NODES // 2, 2 * D)
    s4 = _mean4(flat(x0), flat(x1), flat(x2), flat(x3))

    out = s4.reshape(NC, N_NODES, DH).transpose(1, 0, 2).reshape(N_NODES, D)
    return out[:N_USERS], out[N_USERS:]


# speedup vs baseline: 4.1068x; 4.1068x over previous
import functools

import jax
import jax.numpy as jnp
from jax import lax
from jax.experimental import pallas as pl
from jax.experimental.pallas import tpu as pltpu
from jax.experimental.pallas import tpu_sc as plsc

N_USERS = 25000
N_ITEMS = 25000
N_NODES = N_USERS + N_ITEMS
D = 64
DQ = 16
NQ = D // DQ
NNZ = 800000
N_LAYERS = 3

NS = 16
NC = 2
NROW = 50048
STRIPE = NROW // NS
E_CHUNK = 1024
E_GRP = 128
GRPS = E_CHUNK // E_GRP
CHUNKS_PER_TILE = -(-NNZ // (NS * E_CHUNK))
EDGES_PER_TILE = CHUNKS_PER_TILE * E_CHUNK
NNZ_PAD = EDGES_PER_TILE * NS


_PRE_R = 2000


def _pre_body(t_ref, emb_ref, wa_ref, ba_ref, wp_ref, out_ref):
    ph = t_ref[:] * wa_ref[:] + ba_ref[:]
    lane = lax.broadcasted_iota(jnp.int32, ph.shape, 1)
    val = jnp.where(lane == 0, ph, jnp.sin(ph))
    res = lax.dot_general(val, wp_ref[:], (((1,), (1,)), ((), ())),
                          preferred_element_type=jnp.float32)
    out_ref[:] = res + emb_ref[:]


def _preamble(t_all, emb_all, wa, ba, wproj):
    nb = N_NODES // _PRE_R
    return pl.pallas_call(
        _pre_body,
        grid=(nb,),
        in_specs=[
            pl.BlockSpec((_PRE_R, 1), lambda i: (i, 0)),
            pl.BlockSpec((_PRE_R, D), lambda i: (i, 0)),
            pl.BlockSpec((1, D), lambda i: (0, 0)),
            pl.BlockSpec((1, D), lambda i: (0, 0)),
            pl.BlockSpec((D, D), lambda i: (0, 0)),
        ],
        out_specs=pl.BlockSpec((_PRE_R, D), lambda i: (i, 0)),
        out_shape=jax.ShapeDtypeStruct((N_NODES, D), jnp.float32),
    )(t_all, emb_all, wa, ba, wproj)


def _spmm_body(x0, cols2d, rows2d, vals2d, zstripe,
               o1, o2, o3,
               colv, colv2, rowv, valv, gath, acc, sem):
    c = lax.axis_index("c")
    s = lax.axis_index("s")
    row0 = s * STRIPE
    ebase_row = s * (CHUNKS_PER_TILE * GRPS)

    def zero_stripe():
        pltpu.sync_copy(zstripe, acc.at[pl.ds(row0, STRIPE)])

    def run_pass(src, dst, p):
        col_off = (c * 2 + p) * NROW
        def chunk(k, carry):
            rb = ebase_row + k * GRPS
            pltpu.sync_copy(cols2d.at[pl.ds(rb, GRPS)], colv)
            pltpu.sync_copy(rows2d.at[pl.ds(rb, GRPS)], rowv)
            pltpu.sync_copy(vals2d.at[pl.ds(rb, GRPS)], valv)
            for j in range(GRPS):
                for q in range(E_GRP // 16):
                    colv2[j, pl.ds(q * 16, 16)] = (
                        colv[j, pl.ds(q * 16, 16)] + col_off)
            descs = [pltpu.async_copy(src.at[colv2.at[j]], gath.at[j], sem)
                     for j in range(GRPS)]
            for d in descs:
                d.wait()
            for j in range(GRPS):
                def qbody(q, _, j=j):
                    vv = valv[j, pl.ds(q * 16, 16)]
                    for i in range(16):
                        e = q * 16 + i
                        sv = vv[i]
                        gath[j, e, :] = gath[j, e, :] * sv
                    return 0
                lax.fori_loop(0, E_GRP // 16, qbody, 0)
            for j in range(GRPS):
                pltpu.sync_copy(gath.at[j], acc.at[rowv.at[j]], add=True)
            return carry

        lax.fori_loop(0, CHUNKS_PER_TILE, chunk, 0)
        plsc.subcore_barrier()
        pltpu.sync_copy(acc.at[pl.ds(row0, STRIPE)],
                        dst.at[pl.ds(col_off + row0, STRIPE)])

    first = True
    for src, dst in ((x0, o1), (o1, o2), (o2, o3)):
        for p in range(2):
            zero_stripe()
            plsc.subcore_barrier()
            run_pass(src, dst, p)
            first = False


def _spmm3(x0, cols2d, rows2d, vals2d, zstripe):
    mesh = plsc.VectorSubcoreMesh(core_axis_name="c", subcore_axis_name="s")
    xshape = jax.ShapeDtypeStruct((NQ * NROW, DQ), jnp.float32)
    f = pl.kernel(
        _spmm_body,
        out_type=(xshape, xshape, xshape),
        mesh=mesh,
        scratch_types=[
            pltpu.VMEM((GRPS, E_GRP), jnp.int32),
            pltpu.VMEM((GRPS, E_GRP), jnp.int32),
            pltpu.VMEM((GRPS, E_GRP), jnp.int32),
            pltpu.VMEM((GRPS, E_GRP), jnp.float32),
            pltpu.VMEM((GRPS, E_GRP, DQ), jnp.float32),
            pltpu.VMEM_SHARED((NROW, DQ), jnp.float32),
            pltpu.SemaphoreType.DMA,
        ],
        compiler_params=pltpu.CompilerParams(use_tc_tiling_on_sc=False),
    )
    return f(x0, cols2d, rows2d, vals2d, zstripe)


_M_R = 1000


def _mean_body(a, b, c, d, o):
    o[:] = (a[:] + b[:] + c[:] + d[:]) * 0.25


def _mean4(a, b, c, d):
    nb = a.shape[0] // _M_R
    spec = pl.BlockSpec((_M_R, 128), lambda i: (i, 0))
    return pl.pallas_call(
        _mean_body,
        grid=(nb,),
        in_specs=[spec] * 4,
        out_specs=spec,
        out_shape=jax.ShapeDtypeStruct(a.shape, jnp.float32),
    )(a, b, c, d)


def kernel(user_recency, item_recency, adj_vals, user_emb, item_emb,
           w0, b0, w, b, Wproj, edge_index):
    t_all = jnp.concatenate([user_recency, item_recency]).reshape(N_NODES, 1)
    emb_all = jnp.concatenate([user_emb, item_emb], axis=0)
    wa = jnp.concatenate([w0, w]).reshape(1, D)
    ba = jnp.concatenate([b0, b]).reshape(1, D)

    all_emb = _preamble(t_all, emb_all, wa, ba, Wproj)

    x0 = jnp.transpose(all_emb.reshape(N_NODES, NQ, DQ), (1, 0, 2))
    x0 = jnp.pad(x0, ((0, 0), (0, NROW - N_NODES), (0, 0)))
    x0 = x0.reshape(NQ * NROW, DQ)

    rows = edge_index[0].astype(jnp.int32)
    cols = edge_index[1].astype(jnp.int32)
    pad = NNZ_PAD - NNZ
    ipad = jnp.zeros((pad,), jnp.int32)
    rows2d = jnp.concatenate([rows, ipad]).reshape(-1, E_GRP)
    cols2d = jnp.concatenate([cols, ipad]).reshape(-1, E_GRP)
    vals2d = jnp.concatenate([adj_vals, jnp.zeros((pad,), jnp.float32)])
    vals2d = vals2d.reshape(-1, E_GRP)
    zstripe = jnp.zeros((STRIPE, DQ), jnp.float32)

    x1, x2, x3 = _spmm3(x0, cols2d, rows2d, vals2d, zstripe)

    flat = lambda v: v.reshape(NQ, NROW, DQ)[:, :N_NODES].reshape(
        N_NODES // 2, 2 * D)
    s4 = _mean4(flat(x0), flat(x1), flat(x2), flat(x3))

    out = s4.reshape(NQ, N_NODES, DQ).transpose(1, 0, 2).reshape(N_NODES, D)
    return out[:N_USERS], out[N_USERS:]
